# baseline (speedup 1.0000x reference)
import jax
import jax.numpy as jnp
from jax.experimental import pallas as pl
from jax.experimental.pallas import tpu as pltpu

HIDDEN = 2048
NUM_EXPERTS = 8
TOP_K = 2
BLOCK = 1024


def _router_block(x_ref, w_ref, logits_ref, rw_ref, idx_ref):
    x = x_ref[...]
    w = w_ref[...]
    logits_t = jax.lax.dot_general(
        w, x, (((0,), (1,)), ((), ())),
        preferred_element_type=jnp.float32)
    logits_ref[...] = logits_t

    sub = jax.lax.broadcasted_iota(jnp.int32, logits_t.shape, 0)
    m1 = jnp.max(logits_t, axis=0, keepdims=True)
    i1 = jnp.min(jnp.where(logits_t == m1, sub, NUM_EXPERTS), axis=0,
                 keepdims=True)
    masked = jnp.where(sub == i1, -jnp.inf, logits_t)
    m2 = jnp.max(masked, axis=0, keepdims=True)
    i2 = jnp.min(jnp.where(masked == m2, sub, NUM_EXPERTS), axis=0,
                 keepdims=True)

    e2 = jnp.exp(m2 - m1)
    denom = 1.0 + e2
    rw_ref[...] = jnp.concatenate([1.0 / denom, e2 / denom], axis=0)
    idx_ref[...] = jnp.concatenate([i1, i2], axis=0)


def kernel(hidden_states, W_gate):
    B, S, H = hidden_states.shape
    T = B * S
    x = hidden_states.reshape(T, H)
    grid = (T // BLOCK,)

    logits_t, rw_t, idx_t = pl.pallas_call(
        _router_block,
        grid=grid,
        in_specs=[
            pl.BlockSpec((BLOCK, H), lambda i: (i, 0)),
            pl.BlockSpec((H, NUM_EXPERTS), lambda i: (0, 0)),
        ],
        out_specs=[
            pl.BlockSpec((NUM_EXPERTS, BLOCK), lambda i: (0, i)),
            pl.BlockSpec((TOP_K, BLOCK), lambda i: (0, i)),
            pl.BlockSpec((TOP_K, BLOCK), lambda i: (0, i)),
        ],
        out_shape=[
            jax.ShapeDtypeStruct((NUM_EXPERTS, T), jnp.float32),
            jax.ShapeDtypeStruct((TOP_K, T), jnp.float32),
            jax.ShapeDtypeStruct((TOP_K, T), jnp.int32),
        ],
        compiler_params=pltpu.CompilerParams(
            dimension_semantics=("arbitrary",),
        ),
    )(x, W_gate)

    return (rw_t.T.reshape(B, S, TOP_K),
            idx_t.T.reshape(B, S, TOP_K),
            logits_t.T.reshape(B, S, NUM_EXPERTS))


# speedup vs baseline: 1.0299x; 1.0299x over previous
import jax
import jax.numpy as jnp
from jax.experimental import pallas as pl
from jax.experimental.pallas import tpu as pltpu

HIDDEN = 2048
NUM_EXPERTS = 8
TOP_K = 2
BLOCK = 1024


def _router_block(x_ref, wt_ref, logits_ref, rw_ref, idx_ref):
    x = x_ref[...]
    wt = wt_ref[...]
    logits_t = jax.lax.dot_general(
        wt, x, (((1,), (1,)), ((), ())),
        preferred_element_type=jnp.float32)
    logits_ref[...] = logits_t

    sub = jax.lax.broadcasted_iota(jnp.int32, logits_t.shape, 0)
    m1 = jnp.max(logits_t, axis=0, keepdims=True)
    i1 = jnp.min(jnp.where(logits_t == m1, sub, NUM_EXPERTS), axis=0,
                 keepdims=True)
    masked = jnp.where(sub == i1, -jnp.inf, logits_t)
    m2 = jnp.max(masked, axis=0, keepdims=True)
    i2 = jnp.min(jnp.where(masked == m2, sub, NUM_EXPERTS), axis=0,
                 keepdims=True)

    e2 = jnp.exp(m2 - m1)
    denom = 1.0 + e2
    rw_ref[...] = jnp.concatenate([1.0 / denom, e2 / denom], axis=0)
    idx_ref[...] = jnp.concatenate([i1, i2], axis=0)


def kernel(hidden_states, W_gate):
    B, S, H = hidden_states.shape
    T = B * S
    x = hidden_states.reshape(T, H)
    wt = W_gate.T
    grid = (T // BLOCK,)

    logits_t, rw_t, idx_t = pl.pallas_call(
        _router_block,
        grid=grid,
        in_specs=[
            pl.BlockSpec((BLOCK, H), lambda i: (i, 0)),
            pl.BlockSpec((NUM_EXPERTS, H), lambda i: (0, 0)),
        ],
        out_specs=[
            pl.BlockSpec((NUM_EXPERTS, BLOCK), lambda i: (0, i)),
            pl.BlockSpec((TOP_K, BLOCK), lambda i: (0, i)),
            pl.BlockSpec((TOP_K, BLOCK), lambda i: (0, i)),
        ],
        out_shape=[
            jax.ShapeDtypeStruct((NUM_EXPERTS, T), jnp.float32),
            jax.ShapeDtypeStruct((TOP_K, T), jnp.float32),
            jax.ShapeDtypeStruct((TOP_K, T), jnp.int32),
        ],
        compiler_params=pltpu.CompilerParams(
            dimension_semantics=("parallel",),
        ),
    )(x, wt)

    return (rw_t.T.reshape(B, S, TOP_K),
            idx_t.T.reshape(B, S, TOP_K),
            logits_t.T.reshape(B, S, NUM_EXPERTS))
